# initial kernel scaffold (unmeasured)
import jax
import jax.numpy as jnp
from jax import lax
from jax.experimental import pallas as pl
from jax.experimental.pallas import tpu as pltpu


def kernel(
    x,
):
    def body(*refs):
        pass

    out_shape = jax.ShapeDtypeStruct(..., jnp.float32)
    return pl.pallas_call(body, out_shape=out_shape)(...)


# baseline (device time: 23424 ns/iter reference)
import numpy as np

import jax
import jax.numpy as jnp
from jax import lax
from jax.experimental import pallas as pl
from jax.experimental.pallas import tpu as pltpu

N_DEV = 4


def _bitonic_stages(m):
    stages = []
    k = 2
    while k <= m:
        d = k // 2
        while d >= 1:
            stages.append((k, d))
            d //= 2
        k *= 2
    return stages


def _compare_exchange(val, row_idx, m, k, d):
    up = (row_idx & d) == 0
    asc = (row_idx & k) == 0
    keep_min = asc == up
    shift_up = jnp.concatenate([val[d:], val[:d]], axis=0)
    shift_dn = jnp.concatenate([val[m - d:], val[:m - d]], axis=0)
    partner = jnp.where(up, shift_up, shift_dn)
    mn = jnp.minimum(val, partner)
    mx = jnp.maximum(val, partner)
    return jnp.where(keep_min, mn, mx)


def kernel(x):
    m_per, n = x.shape
    m = N_DEV * m_per
    stages = _bitonic_stages(m)

    def body(x_ref, out_ref, gather_ref, send_sems, recv_sems):
        my_pos = lax.axis_index("i")
        left = lax.rem(my_pos + (N_DEV - 1), N_DEV)
        right = lax.rem(my_pos + 1, N_DEV)

        barrier_sem = pltpu.get_barrier_semaphore()
        for nbr in (left, right):
            pl.semaphore_signal(
                barrier_sem, inc=1,
                device_id=(nbr,), device_id_type=pl.DeviceIdType.MESH,
            )
        pl.semaphore_wait(barrier_sem, 2)

        gather_ref[pl.ds(my_pos * m_per, m_per), :] = x_ref[:, :].astype(
            jnp.bfloat16
        )

        for h in range(N_DEV - 1):
            origin = lax.rem(my_pos + (2 * N_DEV - h), N_DEV)
            rdma = pltpu.make_async_remote_copy(
                src_ref=gather_ref.at[pl.ds(origin * m_per, m_per)],
                dst_ref=gather_ref.at[pl.ds(origin * m_per, m_per)],
                send_sem=send_sems.at[h],
                recv_sem=recv_sems.at[h],
                device_id=(right,),
                device_id_type=pl.DeviceIdType.MESH,
            )
            rdma.start()
            rdma.wait()

        row_idx = lax.broadcasted_iota(jnp.int32, (m, 1), 0)
        val = gather_ref[:, :]
        for k, d in stages:
            val = _compare_exchange(val, row_idx, m, k, d)
        gather_ref[:, :] = val

        out_ref[:, :] = gather_ref[pl.ds(my_pos * m_per, m_per), :].astype(
            jnp.float32
        )

    return pl.pallas_call(
        body,
        out_shape=jax.ShapeDtypeStruct((m_per, n), jnp.float32),
        in_specs=[pl.BlockSpec(memory_space=pltpu.VMEM)],
        out_specs=pl.BlockSpec(memory_space=pltpu.VMEM),
        scratch_shapes=[
            pltpu.VMEM((m, n), jnp.bfloat16),
            pltpu.SemaphoreType.DMA((N_DEV - 1,)),
            pltpu.SemaphoreType.DMA((N_DEV - 1,)),
        ],
        compiler_params=pltpu.CompilerParams(collective_id=0),
    )(x)


# device time: 12737 ns/iter; 1.8391x vs baseline; 1.8391x over previous
import jax
import jax.numpy as jnp
from jax import lax
from jax.experimental import pallas as pl
from jax.experimental.pallas import tpu as pltpu

N_DEV = 4


def _stages(m_lo, m_hi):
    out = []
    k = m_lo
    while k <= m_hi:
        d = k // 2
        while d >= 1:
            out.append((k, d))
            d //= 2
        k *= 2
    return out


def _compare_exchange(val, row_idx, m, k, d, flip=None):
    up = (row_idx & d) == 0
    asc = (row_idx & k) == 0
    keep_min = asc == up
    if flip is not None:
        keep_min = keep_min != flip
    shift_up = jnp.concatenate([val[d:], val[:d]], axis=0)
    shift_dn = jnp.concatenate([val[m - d:], val[:m - d]], axis=0)
    partner = jnp.where(up, shift_up, shift_dn)
    mn = jnp.minimum(val, partner)
    mx = jnp.maximum(val, partner)
    return jnp.where(keep_min, mn, mx)


def kernel(x):
    m_per, n = x.shape
    m = N_DEV * m_per
    local_stages = _stages(2, m_per)
    merge_stages = _stages(2 * m_per, m)

    def body(x_ref, out_ref, gather_ref, send_sems, recv_sems):
        my_pos = lax.axis_index("i")
        is_desc = lax.rem(my_pos, 2) == 1

        barrier_sem = pltpu.get_barrier_semaphore()
        for j in range(1, N_DEV):
            pl.semaphore_signal(
                barrier_sem, inc=1,
                device_id=(lax.rem(my_pos + j, N_DEV),),
                device_id_type=pl.DeviceIdType.MESH,
            )
        pl.semaphore_wait(barrier_sem, N_DEV - 1)

        row_idx_lo = lax.broadcasted_iota(jnp.int32, (m_per, 1), 0)
        v = x_ref[:, :].astype(jnp.bfloat16)
        for k, d in local_stages:
            v = _compare_exchange(v, row_idx_lo, m_per, k, d, flip=is_desc)
        gather_ref[pl.ds(my_pos * m_per, m_per), :] = v

        rdmas = []
        for j in (2, 1, 3):
            rdma = pltpu.make_async_remote_copy(
                src_ref=gather_ref.at[pl.ds(my_pos * m_per, m_per)],
                dst_ref=gather_ref.at[pl.ds(my_pos * m_per, m_per)],
                send_sem=send_sems.at[j - 1],
                recv_sem=recv_sems.at[j - 1],
                device_id=(lax.rem(my_pos + j, N_DEV),),
                device_id_type=pl.DeviceIdType.MESH,
            )
            rdma.start()
            rdmas.append(rdma)
        for rdma in rdmas:
            rdma.wait_recv()

        row_idx = lax.broadcasted_iota(jnp.int32, (m, 1), 0)
        val = gather_ref[:, :]
        for k, d in merge_stages:
            val = _compare_exchange(val, row_idx, m, k, d)
        gather_ref[:, :] = val

        for rdma in rdmas:
            rdma.wait_send()

        out_ref[:, :] = gather_ref[pl.ds(my_pos * m_per, m_per), :].astype(
            jnp.float32
        )

    return pl.pallas_call(
        body,
        out_shape=jax.ShapeDtypeStruct((m_per, n), jnp.float32),
        in_specs=[pl.BlockSpec(memory_space=pltpu.VMEM)],
        out_specs=pl.BlockSpec(memory_space=pltpu.VMEM),
        scratch_shapes=[
            pltpu.VMEM((m, n), jnp.bfloat16),
            pltpu.SemaphoreType.DMA((N_DEV - 1,)),
            pltpu.SemaphoreType.DMA((N_DEV - 1,)),
        ],
        compiler_params=pltpu.CompilerParams(collective_id=0),
    )(x)


# device time: 11067 ns/iter; 2.1166x vs baseline; 1.1509x over previous
import jax
import jax.numpy as jnp
from jax import lax
from jax.experimental import pallas as pl
from jax.experimental.pallas import tpu as pltpu

N_DEV = 4


def _sort_stages(m):
    out = []
    k = 2
    while k <= m:
        d = k // 2
        while d >= 1:
            out.append((k, d))
            d //= 2
        k *= 2
    return out


def _shift_partner(val, m, d):
    shift_up = jnp.concatenate([val[d:], val[:d]], axis=0)
    shift_dn = jnp.concatenate([val[m - d:], val[:m - d]], axis=0)
    return shift_up, shift_dn


def _cex_sort(val, row_idx, m, k, d, flip):
    up = (row_idx & d) == 0
    keep_min = ((row_idx & k) == 0) == up
    keep_min = keep_min != flip
    shift_up, shift_dn = _shift_partner(val, m, d)
    partner = jnp.where(up, shift_up, shift_dn)
    return jnp.where(keep_min, jnp.minimum(val, partner),
                     jnp.maximum(val, partner))


def _cex_merge(val, row_idx, m, d, flip=None):
    up = (row_idx & d) == 0
    keep_min = up if flip is None else up != flip
    shift_up, shift_dn = _shift_partner(val, m, d)
    partner = jnp.where(up, shift_up, shift_dn)
    return jnp.where(keep_min, jnp.minimum(val, partner),
                     jnp.maximum(val, partner))


def kernel(x):
    m_per, n = x.shape
    m = N_DEV * m_per
    m_half = 2 * m_per
    local_stages = _sort_stages(m_per)

    def body(x_ref, out_ref, gather_ref, merged_ref, send_sems, recv_sems):
        my_pos = lax.axis_index("i")
        is_desc = my_pos % 2 == 1
        my_pair = my_pos // 2

        barrier_sem = pltpu.get_barrier_semaphore()
        for j in range(1, N_DEV):
            pl.semaphore_signal(
                barrier_sem, inc=1,
                device_id=(lax.rem(my_pos + j, N_DEV),),
                device_id_type=pl.DeviceIdType.MESH,
            )
        pl.semaphore_wait(barrier_sem, N_DEV - 1)

        idx_per = lax.broadcasted_iota(jnp.int32, (m_per, 1), 0)
        v = x_ref[:, :].astype(jnp.bfloat16)
        for k, d in local_stages:
            v = _cex_sort(v, idx_per, m_per, k, d, flip=is_desc)
        gather_ref[pl.ds(my_pos * m_per, m_per), :] = v

        rdmas = []
        for j in (2, 1, 3):
            rdma = pltpu.make_async_remote_copy(
                src_ref=gather_ref.at[pl.ds(my_pos * m_per, m_per)],
                dst_ref=gather_ref.at[pl.ds(my_pos * m_per, m_per)],
                send_sem=send_sems.at[j - 1],
                recv_sem=recv_sems.at[my_pos],
                device_id=(lax.rem(my_pos + j, N_DEV),),
                device_id_type=pl.DeviceIdType.MESH,
            )
            rdma.start()
            rdmas.append(rdma)

        def wait_chunk(origin):
            pltpu.make_async_remote_copy(
                src_ref=gather_ref.at[pl.ds(origin * m_per, m_per)],
                dst_ref=gather_ref.at[pl.ds(origin * m_per, m_per)],
                send_sem=send_sems.at[0],
                recv_sem=recv_sems.at[origin],
                device_id=(my_pos,),
                device_id_type=pl.DeviceIdType.MESH,
            ).wait_recv()

        idx_half = lax.broadcasted_iota(jnp.int32, (m_half, 1), 0)
        pair_base = my_pair * m_half
        other_base = (1 - my_pair) * m_half

        wait_chunk(my_pos ^ 1)
        pv = gather_ref[pl.ds(pair_base, m_half), :]
        d = m_per
        while d >= 1:
            pv = _cex_merge(pv, idx_half, m_half, d, flip=my_pair == 1)
            d //= 2
        merged_ref[pl.ds(pair_base, m_half), :] = pv

        wait_chunk(my_pos ^ 2)
        wait_chunk(my_pos ^ 3)
        ov = gather_ref[pl.ds(other_base, m_half), :]
        d = m_per
        while d >= 1:
            ov = _cex_merge(ov, idx_half, m_half, d, flip=my_pair == 0)
            d //= 2
        merged_ref[pl.ds(other_base, m_half), :] = ov

        idx_full = lax.broadcasted_iota(jnp.int32, (m, 1), 0)
        val = merged_ref[:, :]
        val = _cex_merge(val, idx_full, m, m_half)
        merged_ref[:, :] = val
        val = merged_ref[pl.ds(pair_base, m_half), :]
        val = _cex_merge(val, idx_half, m_half, m_per)
        merged_ref[pl.ds(pair_base, m_half), :] = val
        val = merged_ref[pl.ds(my_pos * m_per, m_per), :]
        d = m_per // 2
        while d >= 1:
            val = _cex_merge(val, idx_per, m_per, d)
            d //= 2

        for rdma in rdmas:
            rdma.wait_send()

        out_ref[:, :] = val.astype(jnp.float32)

    return pl.pallas_call(
        body,
        out_shape=jax.ShapeDtypeStruct((m_per, n), jnp.float32),
        in_specs=[pl.BlockSpec(memory_space=pltpu.VMEM)],
        out_specs=pl.BlockSpec(memory_space=pltpu.VMEM),
        scratch_shapes=[
            pltpu.VMEM((m, n), jnp.bfloat16),
            pltpu.VMEM((m, n), jnp.bfloat16),
            pltpu.SemaphoreType.DMA((N_DEV - 1,)),
            pltpu.SemaphoreType.DMA((N_DEV,)),
        ],
        compiler_params=pltpu.CompilerParams(collective_id=0),
    )(x)


# device time: 8967 ns/iter; 2.6122x vs baseline; 1.2342x over previous
import jax
import jax.numpy as jnp
from jax import lax
from jax.experimental import pallas as pl
from jax.experimental.pallas import tpu as pltpu

N_DEV = 4


def _sort_stages(m):
    out = []
    k = 2
    while k <= m:
        d = k // 2
        while d >= 1:
            out.append((k, d))
            d //= 2
        k *= 2
    return out


def _partner(val, m, n, d):
    if d >= 16:
        y = val.reshape(m // (2 * d), 2, d, n)
        sw = jnp.concatenate([y[:, 1:2], y[:, 0:1]], axis=1)
        return sw.reshape(m, n)
    row_idx = lax.broadcasted_iota(jnp.int32, (m, 1), 0)
    up = (row_idx & d) == 0
    shift_up = jnp.concatenate([val[d:], val[:d]], axis=0)
    shift_dn = jnp.concatenate([val[m - d:], val[:m - d]], axis=0)
    return jnp.where(up, shift_up, shift_dn)


def _cex(val, m, n, k, d, flip=None):
    row_idx = lax.broadcasted_iota(jnp.int32, (m, 1), 0)
    up = (row_idx & d) == 0
    keep_min = up if k is None else ((row_idx & k) == 0) == up
    if flip is not None:
        keep_min = keep_min != flip
    p = _partner(val, m, n, d)
    return jnp.where(keep_min, jnp.minimum(val, p), jnp.maximum(val, p))


def kernel(x):
    m_per, n = x.shape
    m = N_DEV * m_per
    m_half = 2 * m_per
    local_stages = _sort_stages(m_per)

    def body(x_ref, out_ref, gather_ref, merged_ref, send_sems, recv_sems):
        my_pos = lax.axis_index("i")
        is_desc = my_pos % 2 == 1
        my_pair = my_pos // 2

        barrier_sem = pltpu.get_barrier_semaphore()
        for j in range(1, N_DEV):
            pl.semaphore_signal(
                barrier_sem, inc=1,
                device_id=(lax.rem(my_pos + j, N_DEV),),
                device_id_type=pl.DeviceIdType.MESH,
            )

        v = x_ref[:, :].astype(jnp.bfloat16)
        for k, d in local_stages:
            v = _cex(v, m_per, n, k, d, flip=is_desc)
        gather_ref[pl.ds(my_pos * m_per, m_per), :] = v

        pl.semaphore_wait(barrier_sem, N_DEV - 1)

        rdmas = []
        for idx, target in enumerate((my_pos ^ 1, my_pos ^ 2, my_pos ^ 3)):
            rdma = pltpu.make_async_remote_copy(
                src_ref=gather_ref.at[pl.ds(my_pos * m_per, m_per)],
                dst_ref=gather_ref.at[pl.ds(my_pos * m_per, m_per)],
                send_sem=send_sems.at[idx],
                recv_sem=recv_sems.at[my_pos],
                device_id=(target,),
                device_id_type=pl.DeviceIdType.MESH,
            )
            rdma.start()
            rdmas.append(rdma)

        def wait_chunk(origin):
            pltpu.make_async_remote_copy(
                src_ref=gather_ref.at[pl.ds(origin * m_per, m_per)],
                dst_ref=gather_ref.at[pl.ds(origin * m_per, m_per)],
                send_sem=send_sems.at[0],
                recv_sem=recv_sems.at[origin],
                device_id=(my_pos,),
                device_id_type=pl.DeviceIdType.MESH,
            ).wait_recv()

        pair_base = my_pair * m_half
        other_base = (1 - my_pair) * m_half

        wait_chunk(my_pos ^ 1)
        pv = gather_ref[pl.ds(pair_base, m_half), :]
        d = m_per
        while d >= 1:
            pv = _cex(pv, m_half, n, None, d, flip=my_pair == 1)
            d //= 2
        merged_ref[pl.ds(pair_base, m_half), :] = pv

        wait_chunk(my_pos ^ 2)
        wait_chunk(my_pos ^ 3)
        ov = gather_ref[pl.ds(other_base, m_half), :]
        d = m_per
        while d >= 1:
            ov = _cex(ov, m_half, n, None, d, flip=my_pair == 0)
            d //= 2
        merged_ref[pl.ds(other_base, m_half), :] = ov

        val = merged_ref[:, :]
        val = _cex(val, m, n, None, m_half)
        merged_ref[:, :] = val
        val = merged_ref[pl.ds(pair_base, m_half), :]
        val = _cex(val, m_half, n, None, m_per)
        merged_ref[pl.ds(pair_base, m_half), :] = val
        val = merged_ref[pl.ds(my_pos * m_per, m_per), :]
        d = m_per // 2
        while d >= 1:
            val = _cex(val, m_per, n, None, d)
            d //= 2

        for rdma in rdmas:
            rdma.wait_send()

        out_ref[:, :] = val.astype(jnp.float32)

    return pl.pallas_call(
        body,
        out_shape=jax.ShapeDtypeStruct((m_per, n), jnp.float32),
        in_specs=[pl.BlockSpec(memory_space=pltpu.VMEM)],
        out_specs=pl.BlockSpec(memory_space=pltpu.VMEM),
        scratch_shapes=[
            pltpu.VMEM((m, n), jnp.bfloat16),
            pltpu.VMEM((m, n), jnp.bfloat16),
            pltpu.SemaphoreType.DMA((N_DEV - 1,)),
            pltpu.SemaphoreType.DMA((N_DEV,)),
        ],
        compiler_params=pltpu.CompilerParams(collective_id=0),
    )(x)
